# baseline (speedup 1.0000x reference)
import functools

import jax
import jax.numpy as jnp
from jax import lax
from jax.experimental import pallas as pl
from jax.experimental.pallas import tpu as pltpu
from jax.experimental.pallas import tpu_sc as plsc

B, T, D = 16, 2048, 64
N = B * T
NW = 16
PER_W = N // NW
CHUNK = 128
NCHUNK = PER_W // CHUNK
LANES = 16


def _emb_body(idx_hbm, tok_hbm, pos_hbm, out_hbm,
              idx_v, pos0, pos1, buf0, buf1,
              gsem0, gsem1, ssem0, ssem1, psem0, psem1):
    wid = lax.axis_index("s")
    base = wid * PER_W

    bufs = (buf0, buf1)
    poss = (pos0, pos1)
    gsems = (gsem0, gsem1)
    ssems = (ssem0, ssem1)
    psems = (psem0, psem1)

    pltpu.sync_copy(idx_hbm.at[pl.ds(base, PER_W)], idx_v)

    gathers = [None] * NCHUNK
    stores = [None] * NCHUNK
    poscps = [None] * NCHUNK

    def issue(k):
        b = k % 2
        gathers[k] = pltpu.async_copy(
            tok_hbm.at[idx_v.at[pl.ds(k * CHUNK, CHUNK)]], bufs[b], gsems[b]
        )
        poscps[k] = pltpu.async_copy(
            pos_hbm.at[pl.ds(k * CHUNK, CHUNK)], poss[b], psems[b]
        )

    issue(0)

    for k in range(NCHUNK):
        b = k % 2
        gathers[k].wait()
        poscps[k].wait()
        if k + 1 < NCHUNK:
            if k >= 1:
                stores[k - 1].wait()
            issue(k + 1)

        buf = bufs[b]
        pos_v = poss[b]

        @plsc.parallel_loop(0, CHUNK, unroll=8)
        def add_row(r, buf=buf, pos_v=pos_v):
            for q in range(D // LANES):
                sl = pl.ds(q * LANES, LANES)
                buf[r, sl] = buf[r, sl] + pos_v[r, sl]

        stores[k] = pltpu.async_copy(
            buf, out_hbm.at[pl.ds(base + k * CHUNK, CHUNK)], ssems[b]
        )

    stores[NCHUNK - 2].wait()
    stores[NCHUNK - 1].wait()


@jax.jit
def _emb(idx_flat, tok_table, pos_table):
    mesh = plsc.VectorSubcoreMesh(
        core_axis_name="c", subcore_axis_name="s", num_cores=1)
    return pl.kernel(
        _emb_body,
        out_type=jax.ShapeDtypeStruct((N, D), jnp.float32),
        mesh=mesh,
        scratch_types=[
            pltpu.VMEM((PER_W,), jnp.int32),
            pltpu.VMEM((CHUNK, D), jnp.float32),
            pltpu.VMEM((CHUNK, D), jnp.float32),
            pltpu.VMEM((CHUNK, D), jnp.float32),
            pltpu.VMEM((CHUNK, D), jnp.float32),
            pltpu.SemaphoreType.DMA,
            pltpu.SemaphoreType.DMA,
            pltpu.SemaphoreType.DMA,
            pltpu.SemaphoreType.DMA,
            pltpu.SemaphoreType.DMA,
            pltpu.SemaphoreType.DMA,
        ],
        compiler_params=pltpu.CompilerParams(use_tc_tiling_on_sc=False),
    )(idx_flat, tok_table, pos_table)


def kernel(idx, tok_table, pos_table):
    out = _emb(idx.reshape(N), tok_table, pos_table)
    return out.reshape(B, T, D)


# speedup vs baseline: 1.0139x; 1.0139x over previous
import functools

import jax
import jax.numpy as jnp
from jax import lax
from jax.experimental import pallas as pl
from jax.experimental.pallas import tpu as pltpu
from jax.experimental.pallas import tpu_sc as plsc

B, T, D = 16, 2048, 64
N = B * T
NW = 32
PER_W = N // NW
CHUNK = 128
NCHUNK = PER_W // CHUNK
LANES = 16


def _emb_body(idx_hbm, tok_hbm, pos_hbm, out_hbm,
              idx_v, pos_v, buf0, buf1, gsem0, gsem1, ssem0, ssem1, psem):
    c = lax.axis_index("c")
    s = lax.axis_index("s")
    wid = s * 2 + c
    base = wid * PER_W
    t0 = base % T

    bufs = (buf0, buf1)
    gsems = (gsem0, gsem1)
    ssems = (ssem0, ssem1)

    pltpu.sync_copy(idx_hbm.at[pl.ds(base, PER_W)], idx_v)
    pos_cp = pltpu.async_copy(pos_hbm.at[pl.ds(t0, PER_W)], pos_v, psem)

    gathers = [None] * NCHUNK
    stores = [None] * NCHUNK

    def issue_gather(k):
        b = k % 2
        gathers[k] = pltpu.async_copy(
            tok_hbm.at[idx_v.at[pl.ds(k * CHUNK, CHUNK)]], bufs[b], gsems[b]
        )

    issue_gather(0)
    pos_waited = False

    for k in range(NCHUNK):
        b = k % 2
        gathers[k].wait()
        if k + 1 < NCHUNK:
            if k >= 1:
                stores[k - 1].wait()
            issue_gather(k + 1)
        if not pos_waited:
            pos_cp.wait()
            pos_waited = True

        off = k * CHUNK
        buf = bufs[b]

        @plsc.parallel_loop(0, CHUNK, unroll=8)
        def add_row(r, off=off, buf=buf):
            for q in range(D // LANES):
                sl = pl.ds(q * LANES, LANES)
                buf[r, sl] = buf[r, sl] + pos_v[off + r, sl]

        stores[k] = pltpu.async_copy(
            buf, out_hbm.at[base // T, pl.ds(t0 + off, CHUNK)], ssems[b]
        )

    stores[NCHUNK - 2].wait()
    stores[NCHUNK - 1].wait()


@jax.jit
def _emb(idx_flat, tok_table, pos_table):
    mesh = plsc.VectorSubcoreMesh(core_axis_name="c", subcore_axis_name="s")
    return pl.kernel(
        _emb_body,
        out_type=jax.ShapeDtypeStruct((B, T, D), jnp.float32),
        mesh=mesh,
        scratch_types=[
            pltpu.VMEM((PER_W,), jnp.int32),
            pltpu.VMEM((PER_W, D), jnp.float32),
            pltpu.VMEM((CHUNK, D), jnp.float32),
            pltpu.VMEM((CHUNK, D), jnp.float32),
            pltpu.SemaphoreType.DMA,
            pltpu.SemaphoreType.DMA,
            pltpu.SemaphoreType.DMA,
            pltpu.SemaphoreType.DMA,
            pltpu.SemaphoreType.DMA,
        ],
        compiler_params=pltpu.CompilerParams(
            use_tc_tiling_on_sc=False, skip_device_barrier=True),
    )(idx_flat, tok_table, pos_table)


def kernel(idx, tok_table, pos_table):
    return _emb(idx.reshape(N), tok_table, pos_table)
